# Initial kernel scaffold; baseline (speedup 1.0000x reference)
import jax
import jax.numpy as jnp
from jax.experimental import pallas as pl


def kernel(hidden_states, W_qkv, W_b, W_a, conv_w, W_out, norm_w, A_log, dt_bias):
    raise NotImplementedError("write your pallas kernel here")


# speedup vs baseline: 12.9311x; 12.9311x over previous
import functools

import jax
import jax.numpy as jnp
from jax import lax
from jax.experimental import pallas as pl
from jax.experimental.pallas import tpu as pltpu

H = 2560
DI = 8192
G = 32
DG = 4096
KW = 4
DH = 128
EPS = 1e-6
Q = 256
QE = Q + 8

BL = 8192
NC = 4096 // Q

VMEM_LIMIT = 56 * 1024 * 1024
K1_DTYPE = jnp.float32
K3_DTYPE = jnp.bfloat16
XZ_DTYPE = jnp.float32
YG_DTYPE = jnp.bfloat16
SCAN_MM_DTYPE = jnp.float32


def _dot_tt(a, b):
    return lax.dot_general(a, b, (((1,), (1,)), ((), ())),
                           preferred_element_type=jnp.float32)


def _proj_body(x_ref, w_ref, wba_ref, xz_ref, bc_ref):
    j = pl.program_id(1)
    x = x_ref[...]
    xz_ref[...] = _dot_tt(x, w_ref[...]).astype(xz_ref.dtype)

    @pl.when(j == 0)
    def _():
        bc_ref[...] = _dot_tt(x, wba_ref[...])


def _proj_call(hs_bf, wq_bf, wba_bf, *, interpret=False):
    TM, TN = (2048, 1024) if K1_DTYPE == jnp.bfloat16 else (1024, 512)
    grid = (BL // TM, DI // TN)
    return pl.pallas_call(
        _proj_body,
        grid=grid,
        in_specs=[
            pl.BlockSpec((TM, H), lambda i, j: (i, 0)),
            pl.BlockSpec((TN, H), lambda i, j: (j, 0)),
            pl.BlockSpec((64, H), lambda i, j: (0, 0)),
        ],
        out_specs=[
            pl.BlockSpec((TM, TN), lambda i, j: (i, j)),
            pl.BlockSpec((TM, 64), lambda i, j: (i, 0)),
        ],
        out_shape=[
            jax.ShapeDtypeStruct((BL, DI), XZ_DTYPE),
            jax.ShapeDtypeStruct((BL, 64), jnp.float32),
        ],
        compiler_params=pltpu.CompilerParams(
            dimension_semantics=("parallel", "arbitrary"),
            vmem_limit_bytes=VMEM_LIMIT,
        ),
        name="ssm_proj",
        interpret=interpret,
    )(hs_bf, wq_bf, wba_bf)


def _scan_body(xz_ref, bc_ref, t_ref, ap_ref, daq_ref, cw_ref, nw_ref,
               out_ref, h_ref, prev_ref):
    ci = pl.program_id(1)

    @pl.when(ci == 0)
    def _():
        h_ref[...] = jnp.zeros_like(h_ref)
        prev_ref[...] = jnp.zeros_like(prev_ref)

    xz = xz_ref[0].astype(jnp.float32)
    ext = jnp.concatenate([prev_ref[0:KW - 1], xz], axis=0)
    cw = cw_ref[...]
    xc = (cw[0:1] * ext[0:Q] + cw[1:2] * ext[1:Q + 1]
          + cw[2:3] * ext[2:Q + 2] + cw[3:4] * ext[3:Q + 3])
    prev_ref[0:KW - 1] = xz[Q - (KW - 1):Q]

    x = xc[:, :DG]
    x = x * jax.nn.sigmoid(x)
    z = xc[:, DG:]
    gate = z * jax.nn.sigmoid(z)

    bc = bc_ref[0]
    ap = ap_ref[...]
    nw = nw_ref[...]

    for g in range(G):
        sl = slice(g * DH, (g + 1) * DH)
        u = bc[:, g:g + 1] * x[:, sl]
        r = lax.dot_general(
            t_ref[g].astype(SCAN_MM_DTYPE), u.astype(SCAN_MM_DTYPE),
            (((1,), (0,)), ((), ())),
            preferred_element_type=jnp.float32)
        h_old = h_ref[g:g + 1]
        y = bc[:, G + g:G + g + 1] * (r[0:Q] + ap[:, g:g + 1] * h_old)
        h_ref[g:g + 1] = daq_ref[0:1, g:g + 1] * h_old + r[Q:Q + 1]
        ms = jnp.mean(y * y, axis=1, keepdims=True)
        y = y * lax.rsqrt(ms + EPS) * nw
        out_ref[0, :, sl] = (y * gate[:, sl]).astype(out_ref.dtype)


def _scan_call(xz3, bc3, t_ext, a_pow, daq, cw, nw, *, interpret=False):
    Bsz = xz3.shape[0]
    grid = (Bsz, NC)
    return pl.pallas_call(
        _scan_body,
        grid=grid,
        in_specs=[
            pl.BlockSpec((1, Q, DI), lambda b, c: (b, c, 0)),
            pl.BlockSpec((1, Q, 64), lambda b, c: (b, c, 0)),
            pl.BlockSpec((G, QE, Q), lambda b, c: (0, 0, 0)),
            pl.BlockSpec((Q, G), lambda b, c: (0, 0)),
            pl.BlockSpec((1, G), lambda b, c: (0, 0)),
            pl.BlockSpec((KW, DI), lambda b, c: (0, 0)),
            pl.BlockSpec((1, DH), lambda b, c: (0, 0)),
        ],
        out_specs=pl.BlockSpec((1, Q, DG), lambda b, c: (b, c, 0)),
        out_shape=jax.ShapeDtypeStruct((Bsz, 4096, DG), YG_DTYPE),
        scratch_shapes=[
            pltpu.VMEM((G, DH), jnp.float32),
            pltpu.VMEM((8, DI), jnp.float32),
        ],
        compiler_params=pltpu.CompilerParams(
            dimension_semantics=("parallel", "arbitrary"),
            vmem_limit_bytes=VMEM_LIMIT,
        ),
        name="ssm_scan",
        interpret=interpret,
    )(xz3, bc3, t_ext, a_pow, daq, cw, nw)


def _out_body(y_ref, w_ref, o_ref):
    o_ref[...] = _dot_tt(y_ref[...].astype(K3_DTYPE), w_ref[...])


def _out_call(yg, wo_bf, *, interpret=False):
    TM, TN = (1024, 640) if K3_DTYPE == jnp.bfloat16 else (512, 640)
    grid = (H // TN, BL // TM)
    return pl.pallas_call(
        _out_body,
        grid=grid,
        in_specs=[
            pl.BlockSpec((TM, DG), lambda j, i: (i, 0)),
            pl.BlockSpec((TN, DG), lambda j, i: (j, 0)),
        ],
        out_specs=pl.BlockSpec((TM, TN), lambda j, i: (i, j)),
        out_shape=jax.ShapeDtypeStruct((BL, H), jnp.float32),
        compiler_params=pltpu.CompilerParams(
            dimension_semantics=("parallel", "arbitrary"),
            vmem_limit_bytes=VMEM_LIMIT,
        ),
        name="ssm_out",
        interpret=interpret,
    )(yg, wo_bf)


def _run(hidden_states, W_qkv, W_b, W_a, conv_w, W_out, norm_w, A_log,
         dt_bias, *, interpret=False):
    Bsz, L, _ = hidden_states.shape

    hs_bf = hidden_states.reshape(Bsz * L, H).astype(K1_DTYPE)
    wq_bf = W_qkv.astype(K1_DTYPE)
    wba_bf = jnp.concatenate([W_b, W_a], axis=0).astype(K1_DTYPE)
    wo_bf = W_out.astype(K3_DTYPE)

    dt = jax.nn.softplus(dt_bias.astype(jnp.float32))
    ldA = dt * (-jnp.exp(A_log.astype(jnp.float32)))
    i = jnp.arange(Q, dtype=jnp.float32)
    dij = i[:, None] - i[None, :]
    t_mat = jnp.where(dij >= 0, jnp.exp(ldA[:, None, None] * dij), 0.0)
    t_mat = t_mat * dt[:, None, None]
    w_state = dt[:, None] * jnp.exp(ldA[:, None] * (Q - 1 - i)[None, :])
    t_ext = jnp.concatenate(
        [t_mat, w_state[:, None, :], jnp.zeros((G, 7, Q), jnp.float32)],
        axis=1)
    a_pow = jnp.exp(ldA[None, :] * (i[:, None] + 1.0))
    daq = jnp.exp(ldA * Q)[None, :]
    cw = conv_w[:, 0, :].T.astype(jnp.float32)
    nw = norm_w.astype(jnp.float32)[None, :]

    xz, bc = _proj_call(hs_bf, wq_bf, wba_bf, interpret=interpret)
    xz3 = xz.reshape(Bsz, L, DI)
    bc3 = bc.reshape(Bsz, L, 64)
    yg = _scan_call(xz3, bc3, t_ext, a_pow, daq, cw, nw, interpret=interpret)
    out = _out_call(yg.reshape(Bsz * L, DG), wo_bf, interpret=interpret)
    return out.reshape(Bsz, L, H)


def kernel(hidden_states, W_qkv, W_b, W_a, conv_w, W_out, norm_w, A_log,
           dt_bias):
    return _run(hidden_states, W_qkv, W_b, W_a, conv_w, W_out, norm_w,
                A_log, dt_bias)
